# Initial kernel scaffold; baseline (speedup 1.0000x reference)
import jax
import jax.numpy as jnp
from jax.experimental import pallas as pl


def kernel(edge_index, edge_vals, user_emb, item_emb):
    raise NotImplementedError("write your pallas kernel here")


# speedup vs baseline: 5.9009x; 5.9009x over previous
import functools

import jax
import jax.numpy as jnp
from jax import lax
from jax.experimental import pallas as pl
from jax.experimental.pallas import tpu as pltpu
from jax.experimental.pallas import tpu_sc as plsc

NC = 2
NS = 16
LN = 16
G = 128
SB = 4
HALF = 32


def _sc_body(n_total, sb_per_sub, rows_per_sub, zrows,
             src_hbm, dst_hbm, val_hbm, e0lo, e0hi,
             o1lo, o1hi, o2lo, o2hi, o3lo, o3hi,
             acc, src_b, dst_b, val_b, gbuf, gsem):
    c = lax.axis_index("c")
    s = lax.axis_index("s")
    row0 = s * rows_per_sub

    def run_half(tables_in, tables_out):
        for tin, tout in zip(tables_in, tables_out):
            @pl.loop(0, zrows)
            def _(r):
                gbuf[r, pl.ds(0, LN)] = jnp.zeros((LN,), jnp.float32)
                gbuf[r, pl.ds(LN, LN)] = jnp.zeros((LN,), jnp.float32)

            @pl.loop(0, rows_per_sub // zrows)
            def _(k):
                pltpu.sync_copy(gbuf.at[pl.ds(0, zrows)],
                                acc.at[pl.ds(row0 + k * zrows, zrows)])
            plsc.subcore_barrier()

            base_sb = s * sb_per_sub

            @pl.loop(0, sb_per_sub)
            def _(t):
                g0 = (base_sb + t) * SB
                pltpu.sync_copy(src_hbm.at[pl.ds(g0, SB)], src_b)
                pltpu.sync_copy(dst_hbm.at[pl.ds(g0, SB)], dst_b)
                pltpu.sync_copy(val_hbm.at[pl.ds(g0 * G, SB * G)], val_b)
                cps = [
                    pltpu.async_copy(tin.at[src_b.at[j]],
                                     gbuf.at[pl.ds(j * G, G)], gsem)
                    for j in range(SB)
                ]
                for cp in cps:
                    cp.wait()

                @pl.loop(0, SB * G // LN)
                def _(q):
                    base = q * LN
                    vv = val_b[pl.ds(base, LN)]
                    for i in range(LN):
                        v = vv[i]
                        e = base + i
                        gbuf[e, pl.ds(0, LN)] = gbuf[e, pl.ds(0, LN)] * v
                        gbuf[e, pl.ds(LN, LN)] = gbuf[e, pl.ds(LN, LN)] * v

                for j in range(SB):
                    pltpu.sync_copy(gbuf.at[pl.ds(j * G, G)],
                                    acc.at[dst_b.at[j]], add=True)

            plsc.subcore_barrier()

            @pl.loop(0, rows_per_sub // zrows)
            def _(k):
                r0 = row0 + k * zrows
                pltpu.sync_copy(acc.at[pl.ds(r0, zrows)],
                                tout.at[pl.ds(r0, zrows)])
            plsc.subcore_barrier()

    @pl.when(c == 0)
    def _():
        run_half([e0lo, o1lo, o2lo], [o1lo, o2lo, o3lo])

    @pl.when(c == 1)
    def _():
        run_half([e0hi, o1hi, o2hi], [o1hi, o2hi, o3hi])


def _mean_body(a0, a1, a2, a3, b0, b1, b2, b3, o):
    o[:, pl.ds(0, HALF)] = (a0[...] + a1[...] + a2[...] + a3[...]) * 0.25
    o[:, pl.ds(HALF, HALF)] = (b0[...] + b1[...] + b2[...] + b3[...]) * 0.25


def kernel(edge_index, edge_vals, user_emb, item_emb):
    nu = user_emb.shape[0]
    ni = item_emb.shape[0]
    n_total = nu + ni
    e_edges = edge_vals.shape[0]

    src = edge_index[0].astype(jnp.int32)
    dst = edge_index[1].astype(jnp.int32)
    val = edge_vals.astype(jnp.float32)

    chunk = G * SB * NS
    e_pad = ((e_edges + chunk - 1) // chunk) * chunk
    pad = e_pad - e_edges
    if pad:
        src = jnp.concatenate([src, jnp.zeros((pad,), jnp.int32)])
        dst = jnp.concatenate([dst, jnp.zeros((pad,), jnp.int32)])
        val = jnp.concatenate([val, jnp.zeros((pad,), jnp.float32)])
    srcg = src.reshape(e_pad // G, G)
    dstg = dst.reshape(e_pad // G, G)

    n_pad = ((n_total + NS * 8 - 1) // (NS * 8)) * (NS * 8)
    ego = jnp.concatenate([user_emb, item_emb], axis=0)
    if n_pad != n_total:
        ego = jnp.concatenate(
            [ego, jnp.zeros((n_pad - n_total, 2 * HALF), jnp.float32)])
    e0lo = ego[:, :HALF]
    e0hi = ego[:, HALF:]

    sb_per_sub = e_pad // (G * SB * NS)
    rows_per_sub = n_pad // NS
    zrows = 136
    while rows_per_sub % zrows or zrows % 8:
        zrows -= 8

    half_t = jax.ShapeDtypeStruct((n_pad, HALF), jnp.float32)
    mesh = plsc.VectorSubcoreMesh(core_axis_name="c", subcore_axis_name="s")
    sc_call = pl.kernel(
        functools.partial(_sc_body, n_total, sb_per_sub, rows_per_sub, zrows),
        out_type=[half_t] * 6,
        mesh=mesh,
        scratch_types=[
            pltpu.VMEM_SHARED((n_pad, HALF), jnp.float32),
            pltpu.VMEM((SB, G), jnp.int32),
            pltpu.VMEM((SB, G), jnp.int32),
            pltpu.VMEM((SB * G,), jnp.float32),
            pltpu.VMEM((SB * G, HALF), jnp.float32),
            pltpu.SemaphoreType.DMA,
        ],
        compiler_params=pltpu.CompilerParams(use_tc_tiling_on_sc=False),
    )
    o1lo, o1hi, o2lo, o2hi, o3lo, o3hi = sc_call(srcg, dstg, val, e0lo, e0hi)

    br = 2048
    while n_pad % br or br % 8:
        br -= 8
    mean = pl.pallas_call(
        _mean_body,
        grid=(n_pad // br,),
        in_specs=[pl.BlockSpec((br, HALF), lambda i: (i, 0))] * 8,
        out_specs=pl.BlockSpec((br, 2 * HALF), lambda i: (i, 0)),
        out_shape=jax.ShapeDtypeStruct((n_pad, 2 * HALF), jnp.float32),
    )(e0lo, o1lo, o2lo, o3lo, e0hi, o1hi, o2hi, o3hi)

    return mean[:nu], mean[nu:n_total]


# speedup vs baseline: 6.0740x; 1.0293x over previous
import functools

import jax
import jax.numpy as jnp
from jax import lax
from jax.experimental import pallas as pl
from jax.experimental.pallas import tpu as pltpu
from jax.experimental.pallas import tpu_sc as plsc

NC = 2
NS = 16
LN = 16
G = 128
SB = 3
HALF = 32


def _sc_body(sb_per_sub, rows_per_sub, zrows,
             src_hbm, dst_hbm, val_hbm, e0lo, e0hi,
             o1lo, o1hi, o2lo, o2hi, o3lo, o3hi,
             acc, src_a, dst_a, val_a, gbuf_a, sem_a,
             src_b, dst_b, val_b, gbuf_b, sem_b):
    c = lax.axis_index("c")
    s = lax.axis_index("s")
    row0 = s * rows_per_sub
    base = s * sb_per_sub

    def load_edges(t_sb, sbuf, dbuf, vbuf):
        g0 = t_sb * SB
        pltpu.sync_copy(src_hbm.at[pl.ds(g0, SB)], sbuf)
        pltpu.sync_copy(dst_hbm.at[pl.ds(g0, SB)], dbuf)
        pltpu.sync_copy(val_hbm.at[pl.ds(g0 * G, SB * G)], vbuf)

    def issue_gathers(tin, sbuf, gb, sem):
        for j in range(SB):
            pltpu.async_copy(tin.at[sbuf.at[j]], gb.at[pl.ds(j * G, G)], sem)

    def wait_gathers(tin, gb, sem):
        for j in range(SB):
            pltpu.make_async_copy(tin.at[pl.ds(0, G)],
                                  gb.at[pl.ds(j * G, G)], sem).wait()

    def compute_scale(gb, vbuf):
        @pl.loop(0, SB * G // LN)
        def _(q):
            qb = q * LN
            vv = vbuf[pl.ds(qb, LN)]
            for i in range(LN):
                v = vv[i]
                e = qb + i
                gb[e, pl.ds(0, LN)] = gb[e, pl.ds(0, LN)] * v
                gb[e, pl.ds(LN, LN)] = gb[e, pl.ds(LN, LN)] * v

    def scatter_add(gb, dbuf):
        for j in range(SB):
            pltpu.sync_copy(gb.at[pl.ds(j * G, G)],
                            acc.at[dbuf.at[j]], add=True)

    def run_half(tables_in, tables_out):
        for tin, tout in zip(tables_in, tables_out):
            @pl.loop(0, zrows)
            def _(r):
                gbuf_a[r, pl.ds(0, LN)] = jnp.zeros((LN,), jnp.float32)
                gbuf_a[r, pl.ds(LN, LN)] = jnp.zeros((LN,), jnp.float32)

            @pl.loop(0, rows_per_sub // zrows)
            def _(k):
                pltpu.sync_copy(gbuf_a.at[pl.ds(0, zrows)],
                                acc.at[pl.ds(row0 + k * zrows, zrows)])
            plsc.subcore_barrier()

            load_edges(base, src_a, dst_a, val_a)
            issue_gathers(tin, src_a, gbuf_a, sem_a)

            @pl.loop(0, sb_per_sub // 2)
            def _(t2):
                t = base + t2 * 2
                load_edges(t + 1, src_b, dst_b, val_b)
                wait_gathers(tin, gbuf_a, sem_a)
                issue_gathers(tin, src_b, gbuf_b, sem_b)
                compute_scale(gbuf_a, val_a)
                scatter_add(gbuf_a, dst_a)
                load_edges(t + 2, src_a, dst_a, val_a)
                wait_gathers(tin, gbuf_b, sem_b)
                issue_gathers(tin, src_a, gbuf_a, sem_a)
                compute_scale(gbuf_b, val_b)
                scatter_add(gbuf_b, dst_b)

            wait_gathers(tin, gbuf_a, sem_a)
            plsc.subcore_barrier()

            @pl.loop(0, rows_per_sub // zrows)
            def _(k):
                r0 = row0 + k * zrows
                pltpu.sync_copy(acc.at[pl.ds(r0, zrows)],
                                tout.at[pl.ds(r0, zrows)])
            plsc.subcore_barrier()

    @pl.when(c == 0)
    def _():
        run_half([e0lo, o1lo, o2lo], [o1lo, o2lo, o3lo])

    @pl.when(c == 1)
    def _():
        run_half([e0hi, o1hi, o2hi], [o1hi, o2hi, o3hi])


def _mean_body(a0, a1, a2, a3, b0, b1, b2, b3, o):
    o[:, pl.ds(0, HALF)] = (a0[...] + a1[...] + a2[...] + a3[...]) * 0.25
    o[:, pl.ds(HALF, HALF)] = (b0[...] + b1[...] + b2[...] + b3[...]) * 0.25


def kernel(edge_index, edge_vals, user_emb, item_emb):
    nu = user_emb.shape[0]
    ni = item_emb.shape[0]
    n_total = nu + ni
    e_edges = edge_vals.shape[0]

    src = edge_index[0].astype(jnp.int32)
    dst = edge_index[1].astype(jnp.int32)
    val = edge_vals.astype(jnp.float32)

    unit = G * SB * NS
    sb_per_sub = (e_edges + unit - 1) // unit
    sb_per_sub += sb_per_sub % 2
    e_pad = sb_per_sub * unit + G * SB
    pad = e_pad - e_edges
    if pad:
        src = jnp.concatenate([src, jnp.zeros((pad,), jnp.int32)])
        dst = jnp.concatenate([dst, jnp.zeros((pad,), jnp.int32)])
        val = jnp.concatenate([val, jnp.zeros((pad,), jnp.float32)])
    srcg = src.reshape(e_pad // G, G)
    dstg = dst.reshape(e_pad // G, G)

    n_pad = ((n_total + NS * 8 - 1) // (NS * 8)) * (NS * 8)
    ego = jnp.concatenate([user_emb, item_emb], axis=0)
    if n_pad != n_total:
        ego = jnp.concatenate(
            [ego, jnp.zeros((n_pad - n_total, 2 * HALF), jnp.float32)])
    e0lo = ego[:, :HALF]
    e0hi = ego[:, HALF:]

    rows_per_sub = n_pad // NS
    zrows = 136
    while rows_per_sub % zrows or zrows % 8:
        zrows -= 8

    half_t = jax.ShapeDtypeStruct((n_pad, HALF), jnp.float32)
    mesh = plsc.VectorSubcoreMesh(core_axis_name="c", subcore_axis_name="s")
    dbuf_types = [
        pltpu.VMEM((SB, G), jnp.int32),
        pltpu.VMEM((SB, G), jnp.int32),
        pltpu.VMEM((SB * G,), jnp.float32),
        pltpu.VMEM((SB * G, HALF), jnp.float32),
        pltpu.SemaphoreType.DMA,
    ]
    sc_call = pl.kernel(
        functools.partial(_sc_body, sb_per_sub, rows_per_sub, zrows),
        out_type=[half_t] * 6,
        mesh=mesh,
        scratch_types=[pltpu.VMEM_SHARED((n_pad, HALF), jnp.float32)]
        + dbuf_types + dbuf_types,
        compiler_params=pltpu.CompilerParams(use_tc_tiling_on_sc=False),
    )
    o1lo, o1hi, o2lo, o2hi, o3lo, o3hi = sc_call(srcg, dstg, val, e0lo, e0hi)

    br = 2048
    while n_pad % br or br % 8:
        br -= 8
    mean = pl.pallas_call(
        _mean_body,
        grid=(n_pad // br,),
        in_specs=[pl.BlockSpec((br, HALF), lambda i: (i, 0))] * 8,
        out_specs=pl.BlockSpec((br, 2 * HALF), lambda i: (i, 0)),
        out_shape=jax.ShapeDtypeStruct((n_pad, 2 * HALF), jnp.float32),
    )(e0lo, o1lo, o2lo, o3lo, e0hi, o1hi, o2hi, o3hi)

    return mean[:nu], mean[nu:n_total]
